# baseline (speedup 1.0000x reference)
import functools

import jax
import jax.numpy as jnp
from jax import lax
from jax.experimental import pallas as pl
from jax.experimental.pallas import tpu as pltpu
from jax.experimental.pallas import tpu_sc as plsc

B = 16384
D = 64
DP = 128
NS = 16
NW = NS
B_PER_W = B // NW
CHUNK = 256
N_CHUNKS = B_PER_W // CHUNK


def _make_sc_gather():
    mesh = plsc.VectorSubcoreMesh(
        core_axis_name="c", subcore_axis_name="s", num_cores=1
    )

    @functools.partial(
        pl.kernel,
        mesh=mesh,
        out_type=jax.ShapeDtypeStruct((B, DP), jnp.float32),
        scratch_types=[
            pltpu.VMEM((B_PER_W,), jnp.int32),
            pltpu.VMEM((CHUNK, DP), jnp.float32),
            pltpu.VMEM((CHUNK, DP), jnp.float32),
            pltpu.SemaphoreType.DMA,
            pltpu.SemaphoreType.DMA,
        ],
    )
    def gather_kernel(idx_hbm, table_hbm, out_hbm, idx_v, rows_a, rows_b, sem_a, sem_b):
        wid = lax.axis_index("s")
        base = wid * B_PER_W
        pltpu.sync_copy(idx_hbm.at[pl.ds(base, B_PER_W)], idx_v)
        bufs = [(rows_a, sem_a), (rows_b, sem_b)]
        copies = []
        for j in range(N_CHUNKS):
            rv, sm = bufs[j % 2]
            copies.append(
                pltpu.async_copy(
                    table_hbm.at[idx_v.at[pl.ds(j * CHUNK, CHUNK)]], rv, sm
                )
            )
            if j >= 1:
                copies[j - 1].wait()
                pv, _ = bufs[(j - 1) % 2]
                pltpu.sync_copy(
                    pv, out_hbm.at[pl.ds(base + (j - 1) * CHUNK, CHUNK)]
                )
        copies[N_CHUNKS - 1].wait()
        lv, _ = bufs[(N_CHUNKS - 1) % 2]
        pltpu.sync_copy(
            lv, out_hbm.at[pl.ds(base + (N_CHUNKS - 1) * CHUNK, CHUNK)]
        )

    return gather_kernel


_sc_gather = _make_sc_gather()


def _item_body(x_ref, w_ref, b_ref, o_ref):
    acc = jnp.dot(x_ref[...], w_ref[...], preferred_element_type=jnp.float32)
    o_ref[...] = jnp.maximum(acc + b_ref[...], 0.0)


ROWS_BLK = 1024


def _item_linear(item_x, W_item, b_item):
    return pl.pallas_call(
        _item_body,
        grid=(B // ROWS_BLK,),
        in_specs=[
            pl.BlockSpec((ROWS_BLK, 128), lambda i: (i, 0)),
            pl.BlockSpec((128, D), lambda i: (0, 0)),
            pl.BlockSpec((1, D), lambda i: (0, 0)),
        ],
        out_specs=pl.BlockSpec((ROWS_BLK, D), lambda i: (i, 0)),
        out_shape=jax.ShapeDtypeStruct((B, D), jnp.float32),
    )(item_x, W_item, b_item)


def kernel(user_idx, item_x, emb_table, W_item, b_item):
    idx = user_idx.astype(jnp.int32)
    table2 = emb_table.reshape(-1, DP)
    pairs = _sc_gather(lax.shift_right_logical(idx, 1), table2)
    half = (idx & 1).astype(bool)[:, None]
    hid_user = jnp.where(half, pairs[:, D:], pairs[:, :D])
    hid_item = _item_linear(item_x, W_item, b_item.reshape(1, D))
    return (hid_user, hid_item)
